# baseline (speedup 1.0000x reference)
import functools

import jax
import jax.numpy as jnp
from jax import lax
from jax.experimental import pallas as pl
from jax.experimental.pallas import tpu as pltpu
from jax.experimental.pallas import tpu_sc as plsc

T, B, D = 50, 16384, 16
NE = 8192
NC, NS, L = 2, 16, 16
NW = NC * NS
CHUNK = B // NW
CW = 128
NCH = CHUNK // CW
LISTCAP = CHUNK + L

_mesh = plsc.VectorSubcoreMesh(core_axis_name="c", subcore_axis_name="s")


@functools.partial(
    pl.kernel,
    out_type=(
        jax.ShapeDtypeStruct((B,), jnp.float32),
        jax.ShapeDtypeStruct((T * B, D), jnp.float32),
    ),
    mesh=_mesh,
    compiler_params=pltpu.CompilerParams(
        needs_layout_passes=False, use_tc_tiling_on_sc=False
    ),
    scratch_types=[
        pltpu.VMEM((B,), jnp.int32),
        pltpu.VMEM((NE,), jnp.int32),
        pltpu.VMEM((B,), jnp.float32),
        pltpu.VMEM((B,), jnp.float32),
        pltpu.VMEM((CHUNK,), jnp.int32),
        pltpu.VMEM((CHUNK,), jnp.float32),
        pltpu.VMEM((LISTCAP,), jnp.int32),
        pltpu.VMEM((LISTCAP,), jnp.int32),
        pltpu.VMEM((LISTCAP,), jnp.int32),
        pltpu.VMEM((LISTCAP,), jnp.int32),
        pltpu.VMEM((NCH, CW), jnp.int32),
        pltpu.VMEM((NCH, CW), jnp.int32),
        pltpu.VMEM((NCH, CW), jnp.int32),
        pltpu.VMEM((NCH, CW), jnp.int32),
        pltpu.VMEM((CW, D), jnp.float32),
        pltpu.SemaphoreType.DMA,
        pltpu.SemaphoreType.DMA,
    ],
)
def _exchange_resample(
    ll_hbm, oll_hbm, means_hbm, omeans_hbm, exch_hbm, rs_hbm,
    outll_hbm, outms_hbm,
    mask_v, exch_v, ll_v, oll_v, rs_v, outll_v,
    src0f, pos0f, src1f, pos1f,
    src0c, pos0c, src1c, pos1c,
    gbuf, gsem, ssem,
):
    wid = lax.axis_index("s") * NC + lax.axis_index("c")
    base = wid * CHUNK

    pltpu.sync_copy(exch_hbm, exch_v)
    pltpu.sync_copy(ll_hbm, ll_v)
    pltpu.sync_copy(oll_hbm, oll_v)
    pltpu.sync_copy(rs_hbm.at[pl.ds(base, CHUNK)], rs_v)

    zeros16 = jnp.zeros((L,), jnp.int32)
    ones16 = jnp.ones((L,), jnp.int32)
    neg16 = jnp.full((L,), -1, jnp.int32)

    def _zero_mask(i, _):
        mask_v[pl.ds(i * L, L)] = zeros16
        return 0

    lax.fori_loop(0, B // L, _zero_mask, 0)

    def _mark(i, _):
        idx = exch_v[pl.ds(i * L, L)]
        plsc.store_scatter(mask_v, [idx], ones16)
        return 0

    lax.fori_loop(0, NE // L, _mark, 0)

    def _init_lists(i, _):
        src0f[pl.ds(i * L, L)] = zeros16
        src1f[pl.ds(i * L, L)] = zeros16
        pos0f[pl.ds(i * L, L)] = neg16
        pos1f[pl.ds(i * L, L)] = neg16
        return 0

    lax.fori_loop(0, LISTCAP // L, _init_lists, 0)

    lane = lax.iota(jnp.int32, L)

    def _partition(k, carry):
        c0, c1 = carry
        src = rs_v[pl.ds(k * L, L)]
        sel = plsc.load_gather(mask_v, [src])
        m1 = sel != 0
        m0 = jnp.logical_not(m1)
        pos = base + k * L + lane
        plsc.store_compressed(src0f.at[pl.ds(c0, L)], src, mask=m0)
        plsc.store_compressed(pos0f.at[pl.ds(c0, L)], pos, mask=m0)
        plsc.store_compressed(src1f.at[pl.ds(c1, L)], src, mask=m1)
        plsc.store_compressed(pos1f.at[pl.ds(c1, L)], pos, mask=m1)
        lla = plsc.load_gather(ll_v, [src])
        llb = plsc.load_gather(oll_v, [src])
        outll_v[pl.ds(k * L, L)] = jnp.where(m1, llb, lla)
        c0 = c0 + jnp.sum(m0.astype(jnp.int32))
        c1 = c1 + jnp.sum(m1.astype(jnp.int32))
        return (c0, c1)

    n0, n1 = lax.fori_loop(
        0, CHUNK // L, _partition, (jnp.int32(0), jnp.int32(0))
    )

    pltpu.sync_copy(outll_v, outll_hbm.at[pl.ds(base, CHUNK)])

    for c in range(NCH):
        for j in range(CW // L):
            f = c * CW + j * L
            src0c[c, pl.ds(j * L, L)] = src0f[pl.ds(f, L)]
            pos0c[c, pl.ds(j * L, L)] = pos0f[pl.ds(f, L)]
            src1c[c, pl.ds(j * L, L)] = src1f[pl.ds(f, L)]
            pos1c[c, pl.ds(j * L, L)] = pos1f[pl.ds(f, L)]

    def _t_step(t, _):
        row0 = t * B
        for n, s2d, p2d, tab in (
            (n0, src0c, pos0c, means_hbm),
            (n1, src1c, pos1c, omeans_hbm),
        ):
            for c in range(NCH):
                @pl.when(c * CW < n)
                def _():
                    pltpu.async_copy(
                        tab.at[pl.ds(row0, B)].at[s2d.at[c]], gbuf, gsem
                    ).wait()
                    pltpu.async_copy(
                        gbuf,
                        outms_hbm.at[pl.ds(row0, B)].at[
                            plsc.Indices(p2d.at[c], ignored_value=-1)
                        ],
                        ssem,
                    ).wait()
        return 0

    lax.fori_loop(0, T, _t_step, 0)


@jax.jit
def kernel(loglik, means, other_loglik, other_means, exch_inds, resample_inds):
    out_ll, out_ms = _exchange_resample(
        loglik,
        other_loglik,
        means.reshape(T * B, D),
        other_means.reshape(T * B, D),
        exch_inds,
        resample_inds,
    )
    return out_ll, out_ms.reshape(T, B, D)


# speedup vs baseline: 3.3736x; 3.3736x over previous
import functools

import jax
import jax.numpy as jnp
from jax import lax
from jax.experimental import pallas as pl
from jax.experimental.pallas import tpu as pltpu
from jax.experimental.pallas import tpu_sc as plsc

T, B, D = 50, 16384, 16
NE = 8192
NC, NS, L = 2, 16, 16
NW = NC * NS
NU = T * D // NW
CHUNK = B // NW
EB = 512

_mesh = plsc.VectorSubcoreMesh(core_axis_name="c", subcore_axis_name="s")


@functools.partial(
    pl.kernel,
    out_type=(
        jax.ShapeDtypeStruct((B,), jnp.float32),
        jax.ShapeDtypeStruct((T, D, B), jnp.float32),
    ),
    mesh=_mesh,
    compiler_params=pltpu.CompilerParams(
        needs_layout_passes=False, use_tc_tiling_on_sc=False
    ),
    scratch_types=[
        pltpu.VMEM((B,), jnp.int32),
        pltpu.VMEM((EB,), jnp.int32),
        pltpu.VMEM((B,), jnp.int32),
        pltpu.VMEM((2, 2 * B), jnp.float32),
        pltpu.VMEM((B,), jnp.float32),
        pltpu.VMEM((CHUNK,), jnp.float32),
        pltpu.SemaphoreType.DMA,
        pltpu.SemaphoreType.DMA,
        pltpu.SemaphoreType.DMA,
    ],
)
def _exchange_resample(
    ll_hbm, oll_hbm, means_hbm, omeans_hbm, exch_hbm, rs_hbm,
    outll_hbm, outms_hbm,
    mask_v, exch_v, comb_v, slab_v, out_v, outll_v,
    gsem0, gsem1, ssem,
):
    wid = lax.axis_index("s") * NC + lax.axis_index("c")

    pltpu.sync_copy(rs_hbm, comb_v)

    zeros16 = jnp.zeros((L,), jnp.int32)
    ones16 = jnp.ones((L,), jnp.int32)

    def _zero_mask(i, _):
        mask_v[pl.ds(i * L, L)] = zeros16
        return 0

    lax.fori_loop(0, B // L, _zero_mask, 0)

    def _mark_chunk(e, _):
        pltpu.sync_copy(exch_hbm.at[pl.ds(e * EB, EB)], exch_v)

        def _mark(i, _):
            idx = exch_v[pl.ds(i * L, L)]
            plsc.store_scatter(mask_v, [idx], ones16)
            return 0

        lax.fori_loop(0, EB // L, _mark, 0)
        return 0

    lax.fori_loop(0, NE // EB, _mark_chunk, 0)

    def _comb(g, _):
        src = comb_v[pl.ds(g * L, L)]
        sel = plsc.load_gather(mask_v, [src])
        comb_v[pl.ds(g * L, L)] = src + sel * B
        return 0

    lax.fori_loop(0, B // L, _comb, 0)

    llbuf = slab_v.at[0]
    pltpu.sync_copy(ll_hbm, llbuf.at[pl.ds(0, B)])
    pltpu.sync_copy(oll_hbm, llbuf.at[pl.ds(B, B)])
    lbase = wid * CHUNK

    def _ll(j, _):
        comb = comb_v[pl.ds(lbase + j * L, L)]
        outll_v[pl.ds(j * L, L)] = plsc.load_gather(llbuf, [comb])
        return 0

    lax.fori_loop(0, CHUNK // L, _ll, 0)
    pltpu.sync_copy(outll_v, outll_hbm.at[pl.ds(lbase, CHUNK)])

    def _in_copy(u, slot):
        t = u // D
        d = u % D
        buf = slab_v.at[slot]
        sem = gsem0 if slot == 0 else gsem1
        ca = pltpu.make_async_copy(means_hbm.at[t, d], buf.at[pl.ds(0, B)], sem)
        cb = pltpu.make_async_copy(
            omeans_hbm.at[t, d], buf.at[pl.ds(B, B)], sem
        )
        return ca, cb

    def _out_copy(u):
        t = u // D
        d = u % D
        return pltpu.make_async_copy(out_v, outms_hbm.at[t, d], ssem)

    ca, cb = _in_copy(wid, 0)
    ca.start()
    cb.start()

    def _unit(k, _):
        u = wid + k * NW
        slot = lax.rem(k, 2)
        for s in (0, 1):
            @pl.when(slot == s)
            def _():
                @pl.when(k + 1 < NU)
                def _():
                    na, nb = _in_copy(u + NW, 1 - s)
                    na.start()
                    nb.start()
                wa, wb = _in_copy(u, s)
                wa.wait()
                wb.wait()
                @pl.when(k >= 1)
                def _():
                    _out_copy(u - NW).wait()
                buf = slab_v.at[s]

                def _g(j, _):
                    comb = comb_v[pl.ds(j * L, L)]
                    out_v[pl.ds(j * L, L)] = plsc.load_gather(buf, [comb])
                    return 0

                lax.fori_loop(0, B // L, _g, 0)
                _out_copy(u).start()
        return 0

    lax.fori_loop(0, NU, _unit, 0)
    _out_copy(wid + (NU - 1) * NW).wait()


@jax.jit
def kernel(loglik, means, other_loglik, other_means, exch_inds, resample_inds):
    out_ll, out_ms = _exchange_resample(
        loglik,
        other_loglik,
        means.transpose(0, 2, 1),
        other_means.transpose(0, 2, 1),
        exch_inds,
        resample_inds,
    )
    return out_ll, out_ms.transpose(0, 2, 1)


# speedup vs baseline: 5.3518x; 1.5864x over previous
import functools

import jax
import jax.numpy as jnp
from jax import lax
from jax.experimental import pallas as pl
from jax.experimental.pallas import tpu as pltpu
from jax.experimental.pallas import tpu_sc as plsc

T, B, D = 50, 16384, 16
NE = 8192
NC, NS, L = 2, 16, 16
NW = NC * NS
NU = T * D // NW
CHUNK = B // NW
EB = 512

_mesh = plsc.VectorSubcoreMesh(core_axis_name="c", subcore_axis_name="s")


@functools.partial(
    pl.kernel,
    out_type=(
        jax.ShapeDtypeStruct((B,), jnp.float32),
        jax.ShapeDtypeStruct((T, D, B), jnp.float32),
    ),
    mesh=_mesh,
    compiler_params=pltpu.CompilerParams(
        needs_layout_passes=False, use_tc_tiling_on_sc=False
    ),
    scratch_types=[
        pltpu.VMEM((B,), jnp.int32),
        pltpu.VMEM((EB,), jnp.int32),
        pltpu.VMEM((B,), jnp.int32),
        pltpu.VMEM((2, 2 * B), jnp.float32),
        pltpu.VMEM((B,), jnp.float32),
        pltpu.VMEM((CHUNK,), jnp.float32),
        pltpu.SemaphoreType.DMA,
        pltpu.SemaphoreType.DMA,
        pltpu.SemaphoreType.DMA,
    ],
)
def _exchange_resample(
    ll_hbm, oll_hbm, means_hbm, omeans_hbm, exch_hbm, rs_hbm,
    outll_hbm, outms_hbm,
    mask_v, exch_v, comb_v, slab_v, out_v, outll_v,
    gsem0, gsem1, ssem,
):
    wid = lax.axis_index("s") * NC + lax.axis_index("c")

    pltpu.sync_copy(rs_hbm, comb_v)

    zeros16 = jnp.zeros((L,), jnp.int32)
    ones16 = jnp.ones((L,), jnp.int32)

    def _zero_mask(i):
        mask_v[pl.ds(i, L)] = zeros16

    plsc.parallel_loop(0, B, L, unroll=4)(_zero_mask)

    def _mark_chunk(e, _):
        pltpu.sync_copy(exch_hbm.at[pl.ds(e * EB, EB)], exch_v)

        def _mark(i):
            idx = exch_v[pl.ds(i, L)]
            plsc.store_scatter(mask_v, [idx], ones16)

        plsc.parallel_loop(0, EB, L, unroll=4)(_mark)
        return 0

    lax.fori_loop(0, NE // EB, _mark_chunk, 0)

    def _comb(g):
        src = comb_v[pl.ds(g, L)]
        sel = plsc.load_gather(mask_v, [src])
        comb_v[pl.ds(g, L)] = src + sel * B

    plsc.parallel_loop(0, B, L, unroll=4)(_comb)

    llbuf = slab_v.at[0]
    pltpu.sync_copy(ll_hbm, llbuf.at[pl.ds(0, B)])
    pltpu.sync_copy(oll_hbm, llbuf.at[pl.ds(B, B)])
    lbase = wid * CHUNK

    def _ll(j):
        comb = comb_v[pl.ds(lbase + j, L)]
        outll_v[pl.ds(j, L)] = plsc.load_gather(llbuf, [comb])

    plsc.parallel_loop(0, CHUNK, L, unroll=4)(_ll)
    pltpu.sync_copy(outll_v, outll_hbm.at[pl.ds(lbase, CHUNK)])

    def _in_copy(u, slot):
        t = u // D
        d = u % D
        buf = slab_v.at[slot]
        sem = gsem0 if slot == 0 else gsem1
        ca = pltpu.make_async_copy(means_hbm.at[t, d], buf.at[pl.ds(0, B)], sem)
        cb = pltpu.make_async_copy(
            omeans_hbm.at[t, d], buf.at[pl.ds(B, B)], sem
        )
        return ca, cb

    def _out_copy(u):
        t = u // D
        d = u % D
        return pltpu.make_async_copy(out_v, outms_hbm.at[t, d], ssem)

    ca, cb = _in_copy(wid, 0)
    ca.start()
    cb.start()

    def _unit(k, _):
        u = wid + k * NW
        slot = lax.rem(k, 2)
        for s in (0, 1):
            @pl.when(slot == s)
            def _():
                @pl.when(k + 1 < NU)
                def _():
                    na, nb = _in_copy(u + NW, 1 - s)
                    na.start()
                    nb.start()
                wa, wb = _in_copy(u, s)
                wa.wait()
                wb.wait()
                @pl.when(k >= 1)
                def _():
                    _out_copy(u - NW).wait()
                buf = slab_v.at[s]

                def _g(j):
                    comb = comb_v[pl.ds(j, L)]
                    out_v[pl.ds(j, L)] = plsc.load_gather(buf, [comb])

                plsc.parallel_loop(0, B, L, unroll=8)(_g)
                _out_copy(u).start()
        return 0

    lax.fori_loop(0, NU, _unit, 0)
    _out_copy(wid + (NU - 1) * NW).wait()


@jax.jit
def kernel(loglik, means, other_loglik, other_means, exch_inds, resample_inds):
    out_ll, out_ms = _exchange_resample(
        loglik,
        other_loglik,
        means.transpose(0, 2, 1),
        other_means.transpose(0, 2, 1),
        exch_inds,
        resample_inds,
    )
    return out_ll, out_ms.transpose(0, 2, 1)


# speedup vs baseline: 12.8302x; 2.3974x over previous
import functools

import jax
import jax.numpy as jnp
from jax import lax
from jax.experimental import pallas as pl
from jax.experimental.pallas import tpu as pltpu
from jax.experimental.pallas import tpu_sc as plsc

T, B, D = 50, 16384, 16
NE = 8192
NC, NS, L = 2, 16, 16
NW = NC * NS
NU = T * D // NW
CHUNK = B // NW
EB = 512
BT, BL = B // 128, 128

_mesh = plsc.VectorSubcoreMesh(core_axis_name="c", subcore_axis_name="s")


@functools.partial(
    pl.kernel,
    out_type=(
        jax.ShapeDtypeStruct((B,), jnp.float32),
        jax.ShapeDtypeStruct((T * 2, BT, 8, BL), jnp.float32),
    ),
    mesh=_mesh,
    compiler_params=pltpu.CompilerParams(
        needs_layout_passes=False, use_tc_tiling_on_sc=False
    ),
    scratch_types=[
        pltpu.VMEM((B,), jnp.int32),
        pltpu.VMEM((EB,), jnp.int32),
        pltpu.VMEM((B,), jnp.int32),
        pltpu.VMEM((2, 2, BT, BL), jnp.float32),
        pltpu.VMEM((BT, BL), jnp.float32),
        pltpu.VMEM((CHUNK,), jnp.float32),
        pltpu.SemaphoreType.DMA,
        pltpu.SemaphoreType.DMA,
        pltpu.SemaphoreType.DMA,
    ],
)
def _exchange_resample(
    ll_hbm, oll_hbm, means_hbm, omeans_hbm, exch_hbm, rs_hbm,
    outll_hbm, outms_hbm,
    mask_v, exch_v, comb_v, slab_v, out_v, outll_v,
    gsem0, gsem1, ssem,
):
    wid = lax.axis_index("s") * NC + lax.axis_index("c")

    pltpu.sync_copy(rs_hbm, comb_v)

    zeros16 = jnp.zeros((L,), jnp.int32)
    ones16 = jnp.ones((L,), jnp.int32)

    def _zero_mask(i):
        mask_v[pl.ds(i, L)] = zeros16

    plsc.parallel_loop(0, B, L, unroll=4)(_zero_mask)

    def _mark_chunk(e, _):
        pltpu.sync_copy(exch_hbm.at[pl.ds(e * EB, EB)], exch_v)

        def _mark(i):
            idx = exch_v[pl.ds(i, L)]
            plsc.store_scatter(mask_v, [idx], ones16)

        plsc.parallel_loop(0, EB, L, unroll=4)(_mark)
        return 0

    lax.fori_loop(0, NE // EB, _mark_chunk, 0)

    def _comb(g):
        src = comb_v[pl.ds(g, L)]
        sel = plsc.load_gather(mask_v, [src])
        comb_v[pl.ds(g, L)] = src + sel * B

    plsc.parallel_loop(0, B, L, unroll=4)(_comb)

    def _split(comb):
        return [comb >> 14, (comb >> 7) & 127, comb & 127]

    llbuf = slab_v.at[0]
    pltpu.sync_copy(ll_hbm, llbuf.at[0])
    pltpu.sync_copy(oll_hbm, llbuf.at[1])
    lbase = wid * CHUNK

    def _ll(j):
        comb = comb_v[pl.ds(lbase + j, L)]
        outll_v[pl.ds(j, L)] = plsc.load_gather(llbuf, _split(comb))

    plsc.parallel_loop(0, CHUNK, L, unroll=4)(_ll)
    pltpu.sync_copy(outll_v, outll_hbm.at[pl.ds(lbase, CHUNK)])

    def _views(u):
        t = u // D
        d = u % D
        g = t * 2 + d // 8
        dl = d % 8
        return g, dl

    def _in_copy(u, slot):
        g, dl = _views(u)
        sem = gsem0 if slot == 0 else gsem1
        ca = pltpu.make_async_copy(
            means_hbm.at[g, :, dl, :], slab_v.at[slot, 0], sem
        )
        cb = pltpu.make_async_copy(
            omeans_hbm.at[g, :, dl, :], slab_v.at[slot, 1], sem
        )
        return ca, cb

    def _out_copy(u):
        g, dl = _views(u)
        return pltpu.make_async_copy(out_v, outms_hbm.at[g, :, dl, :], ssem)

    ca, cb = _in_copy(wid, 0)
    ca.start()
    cb.start()

    def _unit(k, _):
        u = wid + k * NW
        slot = lax.rem(k, 2)
        for s in (0, 1):
            @pl.when(slot == s)
            def _():
                @pl.when(k + 1 < NU)
                def _():
                    na, nb = _in_copy(u + NW, 1 - s)
                    na.start()
                    nb.start()
                wa, wb = _in_copy(u, s)
                wa.wait()
                wb.wait()
                @pl.when(k >= 1)
                def _():
                    _out_copy(u - NW).wait()
                buf = slab_v.at[s]

                def _g(j):
                    comb = comb_v[pl.ds(j, L)]
                    out_v[j >> 7, pl.ds(j & 127, L)] = plsc.load_gather(
                        buf, _split(comb)
                    )

                plsc.parallel_loop(0, B, L, unroll=8)(_g)
                _out_copy(u).start()
        return 0

    lax.fori_loop(0, NU, _unit, 0)
    _out_copy(wid + (NU - 1) * NW).wait()


@jax.jit
def kernel(loglik, means, other_loglik, other_means, exch_inds, resample_inds):
    def to4d(x):
        return (
            x.transpose(0, 2, 1)
            .reshape(T, 2, 8, BT, BL)
            .transpose(0, 1, 3, 2, 4)
            .reshape(T * 2, BT, 8, BL)
        )

    out_ll, out_ms = _exchange_resample(
        loglik.reshape(BT, BL),
        other_loglik.reshape(BT, BL),
        to4d(means),
        to4d(other_means),
        exch_inds,
        resample_inds,
    )
    ms = (
        out_ms.reshape(T, 2, BT, 8, BL)
        .transpose(0, 1, 3, 2, 4)
        .reshape(T, D, B)
        .transpose(0, 2, 1)
    )
    return out_ll, ms
